# Initial kernel scaffold; baseline (speedup 1.0000x reference)
import jax
import jax.numpy as jnp
from jax.experimental import pallas as pl


def kernel(hidden_states, gate_w, w1, w2, w3, u1, v1, u2, v2, u3, v3):
    raise NotImplementedError("write your pallas kernel here")


# speedup vs baseline: 1.0328x; 1.0328x over previous
import functools

import jax
import jax.numpy as jnp
from jax.experimental import pallas as pl


def _dot_t(a, b):
    return jax.lax.dot_general(
        a, b, (((1,), (1,)), ((), ())), preferred_element_type=jnp.float32
    )


def _router_kernel(x_ref, gw_ref, v1_ref, v3_ref, rl_ref, t1_ref, t3_ref):
    x = x_ref[...]
    rl_ref[...] = _dot_t(x, gw_ref[...])
    t1_ref[...] = _dot_t(x, v1_ref[...])
    t3_ref[...] = _dot_t(x, v3_ref[...])


def _gate_up_kernel(x_ref, w1_ref, w3_ref, u1_ref, u3_ref, t1_ref, t3_ref, h_ref):
    x = x_ref[...]
    gate = _dot_t(x, w1_ref[...]) + _dot_t(t1_ref[...], u1_ref[...])
    up = _dot_t(x, w3_ref[...]) + _dot_t(t3_ref[...], u3_ref[...])
    h_ref[...] = jax.nn.silu(gate) * up


def _t2_kernel(h_ref, v2_ref, t2_ref):
    t2_ref[...] = _dot_t(h_ref[...], v2_ref[...])


def _down_kernel(h_ref, w2_ref, u2_ref, t2_ref, o_ref):
    o_ref[...] = _dot_t(h_ref[...], w2_ref[...]) + _dot_t(t2_ref[...], u2_ref[...])


@functools.partial(jax.jit, static_argnames=())
def kernel(hidden_states, gate_w, w1, w2, w3, u1, v1, u2, v2, u3, v3):
    b, s, d = hidden_states.shape
    T = b * s
    H = d
    F = w1.shape[0]
    R = u1.shape[1]
    E = gate_w.shape[0]
    x = hidden_states.reshape(T, H)

    tM = min(512, T)
    nM = T // tM

    rl, t1, t3 = pl.pallas_call(
        _router_kernel,
        grid=(nM,),
        in_specs=[
            pl.BlockSpec((tM, H), lambda m: (m, 0)),
            pl.BlockSpec((E, H), lambda m: (0, 0)),
            pl.BlockSpec((R, H), lambda m: (0, 0)),
            pl.BlockSpec((R, H), lambda m: (0, 0)),
        ],
        out_specs=[
            pl.BlockSpec((tM, E), lambda m: (m, 0)),
            pl.BlockSpec((tM, R), lambda m: (m, 0)),
            pl.BlockSpec((tM, R), lambda m: (m, 0)),
        ],
        out_shape=[
            jax.ShapeDtypeStruct((T, E), jnp.float32),
            jax.ShapeDtypeStruct((T, R), jnp.float32),
            jax.ShapeDtypeStruct((T, R), jnp.float32),
        ],
    )(x, gate_w, v1, v3)

    tF = min(512, F)
    nF = F // tF
    h = pl.pallas_call(
        _gate_up_kernel,
        grid=(nF, nM),
        in_specs=[
            pl.BlockSpec((tM, H), lambda f, m: (m, 0)),
            pl.BlockSpec((tF, H), lambda f, m: (f, 0)),
            pl.BlockSpec((tF, H), lambda f, m: (f, 0)),
            pl.BlockSpec((tF, R), lambda f, m: (f, 0)),
            pl.BlockSpec((tF, R), lambda f, m: (f, 0)),
            pl.BlockSpec((tM, R), lambda f, m: (m, 0)),
            pl.BlockSpec((tM, R), lambda f, m: (m, 0)),
        ],
        out_specs=pl.BlockSpec((tM, tF), lambda f, m: (m, f)),
        out_shape=jax.ShapeDtypeStruct((T, F), jnp.float32),
    )(x, w1, w3, u1, u3, t1, t3)

    t2 = pl.pallas_call(
        _t2_kernel,
        grid=(nM,),
        in_specs=[
            pl.BlockSpec((tM, F), lambda m: (m, 0)),
            pl.BlockSpec((R, F), lambda m: (0, 0)),
        ],
        out_specs=pl.BlockSpec((tM, R), lambda m: (m, 0)),
        out_shape=jax.ShapeDtypeStruct((T, R), jnp.float32),
    )(h, v2)

    tH = min(512, H)
    nH = H // tH
    out = pl.pallas_call(
        _down_kernel,
        grid=(nH, nM),
        in_specs=[
            pl.BlockSpec((tM, F), lambda hh, m: (m, 0)),
            pl.BlockSpec((tH, F), lambda hh, m: (hh, 0)),
            pl.BlockSpec((tH, R), lambda hh, m: (hh, 0)),
            pl.BlockSpec((tM, R), lambda hh, m: (m, 0)),
        ],
        out_specs=pl.BlockSpec((tM, tH), lambda hh, m: (m, hh)),
        out_shape=jax.ShapeDtypeStruct((T, H), jnp.float32),
    )(h, w2, u2, t2)

    return out.reshape(b, s, d), rl


# speedup vs baseline: 1.0609x; 1.0272x over previous
import jax
import jax.numpy as jnp
from jax.experimental import pallas as pl

_BF = jnp.bfloat16


def _dot_t(a, b):
    return jax.lax.dot_general(
        a, b, (((1,), (1,)), ((), ())), preferred_element_type=jnp.float32
    )


def _stage_a_kernel(x_ref, gw_ref, v1_ref, v3_ref, rl_ref, xb_ref, t1_ref, t3_ref):
    x = x_ref[...]
    rl_ref[...] = _dot_t(x, gw_ref[...])
    xb = x.astype(_BF)
    xb_ref[...] = xb
    t1_ref[...] = _dot_t(xb, v1_ref[...]).astype(_BF)
    t3_ref[...] = _dot_t(xb, v3_ref[...]).astype(_BF)


def _gate_up_kernel(xb_ref, w1_ref, w3_ref, u1_ref, u3_ref, t1_ref, t3_ref, h_ref):
    xb = xb_ref[...]
    gate = _dot_t(xb, w1_ref[...]) + _dot_t(t1_ref[...], u1_ref[...])
    up = _dot_t(xb, w3_ref[...]) + _dot_t(t3_ref[...], u3_ref[...])
    h_ref[...] = (jax.nn.silu(gate) * up).astype(_BF)


def _down_kernel(h_ref, v2_ref, w2_ref, u2_ref, o_ref):
    h = h_ref[...]
    t2 = _dot_t(h, v2_ref[...]).astype(_BF)
    o_ref[...] = _dot_t(h, w2_ref[...]) + _dot_t(t2, u2_ref[...])


def kernel(hidden_states, gate_w, w1, w2, w3, u1, v1, u2, v2, u3, v3):
    b, s, d = hidden_states.shape
    T = b * s
    H = d
    F = w1.shape[0]
    R = u1.shape[1]
    E = gate_w.shape[0]
    x = hidden_states.reshape(T, H)

    w1b, w3b, w2b = w1.astype(_BF), w3.astype(_BF), w2.astype(_BF)
    u1b, u3b, u2b = u1.astype(_BF), u3.astype(_BF), u2.astype(_BF)
    v1b, v3b, v2b = v1.astype(_BF), v3.astype(_BF), v2.astype(_BF)

    tM = min(512, T)
    nM = T // tM

    rl, xb, t1, t3 = pl.pallas_call(
        _stage_a_kernel,
        grid=(nM,),
        in_specs=[
            pl.BlockSpec((tM, H), lambda m: (m, 0)),
            pl.BlockSpec((E, H), lambda m: (0, 0)),
            pl.BlockSpec((R, H), lambda m: (0, 0)),
            pl.BlockSpec((R, H), lambda m: (0, 0)),
        ],
        out_specs=[
            pl.BlockSpec((tM, E), lambda m: (m, 0)),
            pl.BlockSpec((tM, H), lambda m: (m, 0)),
            pl.BlockSpec((tM, R), lambda m: (m, 0)),
            pl.BlockSpec((tM, R), lambda m: (m, 0)),
        ],
        out_shape=[
            jax.ShapeDtypeStruct((T, E), jnp.float32),
            jax.ShapeDtypeStruct((T, H), _BF),
            jax.ShapeDtypeStruct((T, R), _BF),
            jax.ShapeDtypeStruct((T, R), _BF),
        ],
    )(x, gate_w, v1b, v3b)

    tF = min(1024, F)
    nF = F // tF
    h = pl.pallas_call(
        _gate_up_kernel,
        grid=(nF, nM),
        in_specs=[
            pl.BlockSpec((tM, H), lambda f, m: (m, 0)),
            pl.BlockSpec((tF, H), lambda f, m: (f, 0)),
            pl.BlockSpec((tF, H), lambda f, m: (f, 0)),
            pl.BlockSpec((tF, R), lambda f, m: (f, 0)),
            pl.BlockSpec((tF, R), lambda f, m: (f, 0)),
            pl.BlockSpec((tM, R), lambda f, m: (m, 0)),
            pl.BlockSpec((tM, R), lambda f, m: (m, 0)),
        ],
        out_specs=pl.BlockSpec((tM, tF), lambda f, m: (m, f)),
        out_shape=jax.ShapeDtypeStruct((T, F), _BF),
    )(xb, w1b, w3b, u1b, u3b, t1, t3)

    tH = min(1024, H)
    nH = H // tH
    out = pl.pallas_call(
        _down_kernel,
        grid=(nH, nM),
        in_specs=[
            pl.BlockSpec((tM, F), lambda hh, m: (m, 0)),
            pl.BlockSpec((R, F), lambda hh, m: (0, 0)),
            pl.BlockSpec((tH, F), lambda hh, m: (hh, 0)),
            pl.BlockSpec((tH, R), lambda hh, m: (hh, 0)),
        ],
        out_specs=pl.BlockSpec((tM, tH), lambda hh, m: (m, hh)),
        out_shape=jax.ShapeDtypeStruct((T, H), jnp.float32),
    )(h, v2b, w2b, u2b)

    return out.reshape(b, s, d), rl
